# baseline (speedup 1.0000x reference)
import functools

import numpy as np
import jax
import jax.numpy as jnp
from jax import lax
from jax.experimental import pallas as pl
from jax.experimental.pallas import tpu as pltpu
from jax.experimental.pallas import tpu_sc as plsc

B = 4096
NCAT = 26
NCONT = 13
VOCAB = 100000
ED = 32
NEMB = NCAT * ED
L1 = 512
L2 = 256
NCLS = 2
EPS = 1e-5

_NC = 2
_NS = 16
_NW = _NC * _NS


_VMAIN = 99968
_CHUNKS = ((0, 25088), (25088, 25088), (50176, 25088), (75264, 24704))


def _sc_gather_body(t3_hbm, tail_hbm, xcatt_hbm, out_hbm,
                    row_v, tail_v, xf_v, res_v, sem, sem2):
    w = lax.axis_index("s") * _NC + lax.axis_index("c")

    def field_body(c, carry):
        descs = [
            pltpu.make_async_copy(
                t3_hbm.at[c, w, pl.ds(o, l)],
                row_v.at[pl.ds(o, l)],
                sem,
            )
            for o, l in _CHUNKS
        ]
        for d in descs:
            d.start()
        xd = pltpu.make_async_copy(xcatt_hbm.at[c, :], xf_v, sem2)
        td = pltpu.make_async_copy(tail_hbm.at[c, w, :], tail_v, sem2)
        xd.start()
        td.start()
        for d in descs:
            d.wait()
        xd.wait()
        td.wait()

        def group_body(m, carry2):
            for u in range(16):
                off = m * 256 + u * 16
                vi = xf_v[pl.ds(off, 16)].astype(jnp.int32)
                vmain = jnp.minimum(vi, _VMAIN - 1)
                vtail = jnp.maximum(vi - _VMAIN, 0)
                hit = plsc.load_gather(row_v, [vmain])
                tl = plsc.load_gather(tail_v, [vtail])
                res_v[pl.ds(off, 16)] = jnp.where(vi >= _VMAIN, tl, hit)
            return carry2

        lax.fori_loop(0, B // 256, group_body, 0)
        pltpu.sync_copy(res_v, out_hbm.at[c * _NW + w, :])
        return carry

    lax.fori_loop(0, NCAT, field_body, 0)


def _make_sc_gather():
    return functools.partial(
        pl.kernel,
        out_type=jax.ShapeDtypeStruct((NEMB, B), jnp.float32),
        mesh=plsc.VectorSubcoreMesh(core_axis_name="c", subcore_axis_name="s",
                                    num_cores=_NC, num_subcores=_NS),
        scratch_types=[
            pltpu.VMEM((_VMAIN,), jnp.float32),
            pltpu.VMEM((VOCAB - _VMAIN,), jnp.float32),
            pltpu.VMEM((B,), jnp.float32),
            pltpu.VMEM((B,), jnp.float32),
            pltpu.SemaphoreType.DMA,
            pltpu.SemaphoreType.DMA,
        ],
        compiler_params=pltpu.CompilerParams(needs_layout_passes=False),
    )(_sc_gather_body)


def _bn_cols(v, g, b):
    m = jnp.mean(v, axis=0, keepdims=True)
    vm = v - m
    var = jnp.mean(vm * vm, axis=0, keepdims=True)
    return vm * lax.rsqrt(var + EPS) * g + b


def _mlp_body(x1t_ref, xc_ref, w1a_ref, w1b_ref, b1_ref, w2_ref, b2_ref,
              w3_ref, b3_ref, g1_ref, be1_ref, g2_ref, be2_ref, g3_ref,
              be3_ref, out_ref):
    x2 = _bn_cols(xc_ref[:], g1_ref[:], be1_ref[:])
    h = lax.dot_general(x1t_ref[:], w1a_ref[:], (((0,), (0,)), ((), ())),
                        preferred_element_type=jnp.float32)
    h = h + jnp.dot(x2, w1b_ref[:], preferred_element_type=jnp.float32)
    h = jnp.maximum(h + b1_ref[:], 0.0)
    h = _bn_cols(h, g2_ref[:], be2_ref[:])
    h = jnp.dot(h, w2_ref[:], preferred_element_type=jnp.float32)
    h = jnp.maximum(h + b2_ref[:], 0.0)
    h = _bn_cols(h, g3_ref[:], be3_ref[:])
    out_ref[:] = (
        jnp.dot(h, w3_ref[:], preferred_element_type=jnp.float32) + b3_ref[:]
    )


_mlp = pl.pallas_call(
    _mlp_body,
    out_shape=jax.ShapeDtypeStruct((B, NCLS), jnp.float32),
)


def kernel(x, emb_tables, W1, b1, W2, b2, W3, b3, g1, be1, g2, be2, g3, be3):
    t3 = jnp.swapaxes(emb_tables, 1, 2)
    tail = lax.slice(t3, (0, 0, _VMAIN), (NCAT, ED, VOCAB))
    xcatt = x[:, :NCAT].T
    x1t = _make_sc_gather()(t3, tail, xcatt)
    xc = x[:, NCAT:]
    return _mlp(
        x1t, xc, W1[:NEMB], W1[NEMB:], b1.reshape(1, L1), W2,
        b2.reshape(1, L2), W3, b3.reshape(1, NCLS), g1.reshape(1, NCONT),
        be1.reshape(1, NCONT), g2.reshape(1, L1), be2.reshape(1, L1),
        g3.reshape(1, L2), be3.reshape(1, L2),
    )


# speedup vs baseline: 1.3103x; 1.3103x over previous
import functools

import numpy as np
import jax
import jax.numpy as jnp
from jax import lax
from jax.experimental import pallas as pl
from jax.experimental.pallas import tpu as pltpu
from jax.experimental.pallas import tpu_sc as plsc

B = 4096
NCAT = 26
NCONT = 13
VOCAB = 100000
ED = 32
NEMB = NCAT * ED
L1 = 512
L2 = 256
NCLS = 2
EPS = 1e-5

_NC = 2
_NS = 16
_NW = _NC * _NS


_VMAIN = 99968
_ALEN = 50176
_BLEN = _VMAIN - _ALEN
_ACH = ((0, 25088), (25088, 25088))
_BCH = ((50176, 25088), (75264, 24704))


def _sc_gather_body(t3_hbm, tail_hbm, xcatt_hbm, out_hbm,
                    ra_v, rb_v, tail_v, xf_v, res_v, sema, semb, sem2):
    w = lax.axis_index("s") * _NC + lax.axis_index("c")

    def start_a(c):
        for o, l in _ACH:
            pltpu.make_async_copy(t3_hbm.at[c, w, pl.ds(o, l)],
                                  ra_v.at[pl.ds(o, l)], sema).start()

    def start_b(c):
        for o, l in _BCH:
            pltpu.make_async_copy(t3_hbm.at[c, w, pl.ds(o - _ALEN, l)],
                                  rb_v.at[pl.ds(o - _ALEN, l)], semb).start()

    def start_small(c):
        pltpu.make_async_copy(xcatt_hbm.at[c, :], xf_v, sem2).start()
        pltpu.make_async_copy(tail_hbm.at[c, w, :], tail_v, sem2).start()

    def wait_a():
        for o, l in _ACH:
            pltpu.make_async_copy(t3_hbm.at[0, w, pl.ds(o, l)],
                                  ra_v.at[pl.ds(o, l)], sema).wait()

    def wait_b():
        for o, l in _BCH:
            pltpu.make_async_copy(t3_hbm.at[0, w, pl.ds(o - _ALEN, l)],
                                  rb_v.at[pl.ds(o - _ALEN, l)], semb).wait()

    def wait_small():
        pltpu.make_async_copy(xcatt_hbm.at[0, :], xf_v, sem2).wait()
        pltpu.make_async_copy(tail_hbm.at[0, w, :], tail_v, sem2).wait()

    start_a(0)
    start_small(0)
    start_b(0)

    def field_body(c, carry):
        cn = jnp.minimum(c + 1, NCAT - 1)
        wait_small()
        wait_a()

        def pass1(m, carry2):
            for u in range(16):
                off = m * 256 + u * 16
                vi = xf_v[pl.ds(off, 16)].astype(jnp.int32)
                va = jnp.minimum(vi, _ALEN - 1)
                res_v[pl.ds(off, 16)] = plsc.load_gather(ra_v, [va])
            return carry2

        lax.fori_loop(0, B // 256, pass1, 0)
        start_a(cn)
        wait_b()

        def pass2(m, carry2):
            for u in range(16):
                off = m * 256 + u * 16
                vi = xf_v[pl.ds(off, 16)].astype(jnp.int32)
                vb = jnp.clip(vi - _ALEN, 0, _BLEN - 1)
                vt = jnp.clip(vi - _VMAIN, 0, VOCAB - _VMAIN - 1)
                hb = plsc.load_gather(rb_v, [vb])
                ht = plsc.load_gather(tail_v, [vt])
                prev = res_v[pl.ds(off, 16)]
                mid = jnp.where(vi >= _ALEN, hb, prev)
                res_v[pl.ds(off, 16)] = jnp.where(vi >= _VMAIN, ht, mid)
            return carry2

        lax.fori_loop(0, B // 256, pass2, 0)
        pltpu.sync_copy(res_v, out_hbm.at[c * _NW + w, :])
        start_small(cn)
        start_b(cn)
        return carry

    lax.fori_loop(0, NCAT, field_body, 0)
    wait_small()
    wait_a()
    wait_b()


def _make_sc_gather():
    return functools.partial(
        pl.kernel,
        out_type=jax.ShapeDtypeStruct((NEMB, B), jnp.float32),
        mesh=plsc.VectorSubcoreMesh(core_axis_name="c", subcore_axis_name="s",
                                    num_cores=_NC, num_subcores=_NS),
        scratch_types=[
            pltpu.VMEM((_ALEN,), jnp.float32),
            pltpu.VMEM((_BLEN,), jnp.float32),
            pltpu.VMEM((VOCAB - _VMAIN,), jnp.float32),
            pltpu.VMEM((B,), jnp.float32),
            pltpu.VMEM((B,), jnp.float32),
            pltpu.SemaphoreType.DMA,
            pltpu.SemaphoreType.DMA,
            pltpu.SemaphoreType.DMA,
        ],
        compiler_params=pltpu.CompilerParams(needs_layout_passes=False),
    )(_sc_gather_body)


def _bn_cols(v, g, b):
    m = jnp.mean(v, axis=0, keepdims=True)
    vm = v - m
    var = jnp.mean(vm * vm, axis=0, keepdims=True)
    return vm * lax.rsqrt(var + EPS) * g + b


def _mlp_body(x1t_ref, xc_ref, w1a_ref, w1b_ref, b1_ref, w2_ref, b2_ref,
              w3_ref, b3_ref, g1_ref, be1_ref, g2_ref, be2_ref, g3_ref,
              be3_ref, out_ref):
    x2 = _bn_cols(xc_ref[:], g1_ref[:], be1_ref[:])
    h = lax.dot_general(x1t_ref[:], w1a_ref[:], (((0,), (0,)), ((), ())),
                        preferred_element_type=jnp.float32)
    h = h + jnp.dot(x2, w1b_ref[:], preferred_element_type=jnp.float32)
    h = jnp.maximum(h + b1_ref[:], 0.0)
    h = _bn_cols(h, g2_ref[:], be2_ref[:])
    h = jnp.dot(h, w2_ref[:], preferred_element_type=jnp.float32)
    h = jnp.maximum(h + b2_ref[:], 0.0)
    h = _bn_cols(h, g3_ref[:], be3_ref[:])
    out_ref[:] = (
        jnp.dot(h, w3_ref[:], preferred_element_type=jnp.float32) + b3_ref[:]
    )


_mlp = pl.pallas_call(
    _mlp_body,
    out_shape=jax.ShapeDtypeStruct((B, NCLS), jnp.float32),
)


def kernel(x, emb_tables, W1, b1, W2, b2, W3, b3, g1, be1, g2, be2, g3, be3):
    t3 = jnp.swapaxes(emb_tables, 1, 2)
    tail = lax.slice(t3, (0, 0, _VMAIN), (NCAT, ED, VOCAB))
    xcatt = x[:, :NCAT].T
    x1t = _make_sc_gather()(t3, tail, xcatt)
    xc = x[:, NCAT:]
    return _mlp(
        x1t, xc, W1[:NEMB], W1[NEMB:], b1.reshape(1, L1), W2,
        b2.reshape(1, L2), W3, b3.reshape(1, NCLS), g1.reshape(1, NCONT),
        be1.reshape(1, NCONT), g2.reshape(1, L1), be2.reshape(1, L1),
        g3.reshape(1, L2), be3.reshape(1, L2),
    )


# speedup vs baseline: 1.3270x; 1.0127x over previous
import functools

import numpy as np
import jax
import jax.numpy as jnp
from jax import lax
from jax.experimental import pallas as pl
from jax.experimental.pallas import tpu as pltpu
from jax.experimental.pallas import tpu_sc as plsc

B = 4096
NCAT = 26
NCONT = 13
VOCAB = 100000
ED = 32
NEMB = NCAT * ED
L1 = 512
L2 = 256
NCLS = 2
EPS = 1e-5

_NC = 2
_NS = 16
_NW = _NC * _NS


_VMAIN = 99968
_ALEN = 50176
_BLEN = _VMAIN - _ALEN
_ACH = ((0, 25088), (25088, 25088))
_BCH = ((50176, 25088), (75264, 24704))


def _sc_gather_body(t3_hbm, tail_hbm, xcatt_hbm, out_hbm,
                    ra_v, rb_v, tail_v, xf_v, res_v, sema, semb, sem2):
    w = lax.axis_index("s") * _NC + lax.axis_index("c")

    def start_a(c):
        for o, l in _ACH:
            pltpu.make_async_copy(t3_hbm.at[c, w, pl.ds(o, l)],
                                  ra_v.at[pl.ds(o, l)], sema).start()

    def start_b(c):
        for o, l in _BCH:
            pltpu.make_async_copy(t3_hbm.at[c, w, pl.ds(o, l)],
                                  rb_v.at[pl.ds(o - _ALEN, l)], semb).start()

    def start_small(c):
        pltpu.make_async_copy(xcatt_hbm.at[c, :], xf_v, sem2).start()
        pltpu.make_async_copy(tail_hbm.at[c, w, :], tail_v, sem2).start()

    def wait_a():
        for o, l in _ACH:
            pltpu.make_async_copy(t3_hbm.at[0, w, pl.ds(o, l)],
                                  ra_v.at[pl.ds(o, l)], sema).wait()

    def wait_b():
        for o, l in _BCH:
            pltpu.make_async_copy(t3_hbm.at[0, w, pl.ds(o, l)],
                                  rb_v.at[pl.ds(o - _ALEN, l)], semb).wait()

    def wait_small():
        pltpu.make_async_copy(xcatt_hbm.at[0, :], xf_v, sem2).wait()
        pltpu.make_async_copy(tail_hbm.at[0, w, :], tail_v, sem2).wait()

    start_a(0)
    start_small(0)
    start_b(0)

    def field_body(c, carry):
        cn = jnp.minimum(c + 1, NCAT - 1)
        wait_small()
        wait_a()

        def pass1(m, carry2):
            for u in range(16):
                off = m * 256 + u * 16
                vi = xf_v[pl.ds(off, 16)].astype(jnp.int32)
                va = jnp.minimum(vi, _ALEN - 1)
                res_v[pl.ds(off, 16)] = plsc.load_gather(ra_v, [va])
            return carry2

        lax.fori_loop(0, B // 256, pass1, 0)
        start_a(cn)
        wait_b()

        def pass2(m, carry2):
            for u in range(16):
                off = m * 256 + u * 16
                vi = xf_v[pl.ds(off, 16)].astype(jnp.int32)
                vb = jnp.clip(vi - _ALEN, 0, _BLEN - 1)
                vt = jnp.clip(vi - _VMAIN, 0, VOCAB - _VMAIN - 1)
                hb = plsc.load_gather(rb_v, [vb])
                ht = plsc.load_gather(tail_v, [vt])
                prev = res_v[pl.ds(off, 16)]
                mid = jnp.where(vi >= _ALEN, hb, prev)
                res_v[pl.ds(off, 16)] = jnp.where(vi >= _VMAIN, ht, mid)
            return carry2

        lax.fori_loop(0, B // 256, pass2, 0)
        pltpu.sync_copy(res_v, out_hbm.at[c * _NW + w, :])
        start_small(cn)
        start_b(cn)
        return carry

    lax.fori_loop(0, NCAT, field_body, 0)
    wait_small()
    wait_a()
    wait_b()


def _make_sc_gather():
    return functools.partial(
        pl.kernel,
        out_type=jax.ShapeDtypeStruct((NEMB, B), jnp.float32),
        mesh=plsc.VectorSubcoreMesh(core_axis_name="c", subcore_axis_name="s",
                                    num_cores=_NC, num_subcores=_NS),
        scratch_types=[
            pltpu.VMEM((_ALEN,), jnp.float32),
            pltpu.VMEM((_BLEN,), jnp.float32),
            pltpu.VMEM((VOCAB - _VMAIN,), jnp.float32),
            pltpu.VMEM((B,), jnp.float32),
            pltpu.VMEM((B,), jnp.float32),
            pltpu.SemaphoreType.DMA,
            pltpu.SemaphoreType.DMA,
            pltpu.SemaphoreType.DMA,
        ],
        compiler_params=pltpu.CompilerParams(needs_layout_passes=False),
    )(_sc_gather_body)


def _bn_cols(v, g, b):
    m = jnp.mean(v, axis=0, keepdims=True)
    vm = v - m
    var = jnp.mean(vm * vm, axis=0, keepdims=True)
    return vm * lax.rsqrt(var + EPS) * g + b


def _mlp_body(x1t_ref, xc_ref, w1a_ref, w1b_ref, b1_ref, w2_ref, b2_ref,
              w3_ref, b3_ref, g1_ref, be1_ref, g2_ref, be2_ref, g3_ref,
              be3_ref, out_ref):
    x2 = _bn_cols(xc_ref[:], g1_ref[:], be1_ref[:])
    h = lax.dot_general(x1t_ref[:], w1a_ref[:], (((0,), (0,)), ((), ())),
                        preferred_element_type=jnp.float32)
    h = h + jnp.dot(x2, w1b_ref[:], preferred_element_type=jnp.float32)
    h = jnp.maximum(h + b1_ref[:], 0.0)
    h = _bn_cols(h, g2_ref[:], be2_ref[:])
    h = jnp.dot(h, w2_ref[:], preferred_element_type=jnp.float32)
    h = jnp.maximum(h + b2_ref[:], 0.0)
    h = _bn_cols(h, g3_ref[:], be3_ref[:])
    out_ref[:] = (
        jnp.dot(h, w3_ref[:], preferred_element_type=jnp.float32) + b3_ref[:]
    )


_mlp = pl.pallas_call(
    _mlp_body,
    out_shape=jax.ShapeDtypeStruct((B, NCLS), jnp.float32),
)


def kernel(x, emb_tables, W1, b1, W2, b2, W3, b3, g1, be1, g2, be2, g3, be3):
    t3 = jnp.swapaxes(emb_tables, 1, 2)
    tail = lax.slice(t3, (0, 0, _VMAIN), (NCAT, ED, VOCAB))
    xcatt = x[:, :NCAT].T
    x1t = _make_sc_gather()(t3, tail, xcatt)
    xc = x[:, NCAT:]
    return _mlp(
        x1t, xc, W1[:NEMB], W1[NEMB:], b1.reshape(1, L1), W2,
        b2.reshape(1, L2), W3, b3.reshape(1, NCLS), g1.reshape(1, NCONT),
        be1.reshape(1, NCONT), g2.reshape(1, L1), be2.reshape(1, L1),
        g3.reshape(1, L2), be3.reshape(1, L2),
    )
